# baseline (speedup 1.0000x reference)
import functools

import jax
import jax.numpy as jnp
from jax import lax
from jax.experimental import pallas as pl
from jax.experimental.pallas import tpu as pltpu
from jax.experimental.pallas import tpu_sc as plsc

_NC = 2
_NS = 16
_CH = 128


def _sc_mesh():
    return plsc.VectorSubcoreMesh(core_axis_name="c", subcore_axis_name="s")


def _init_slices(N):
    rpt = (N // _NS) & ~7
    tail = N - _NS * rpt
    return rpt, tail


def _build_sc_deg(N, NACC, H, KCHD):
    rpt_acc = NACC // _NS
    rpt, tail = _init_slices(N)

    @functools.partial(
        pl.kernel,
        out_type=jax.ShapeDtypeStruct((_NC, N, H), jnp.float32),
        mesh=_sc_mesh(),
        scratch_types=[
            pltpu.VMEM((KCHD, _CH), jnp.int32),
            pltpu.VMEM((_CH, H), jnp.float32),
            pltpu.VMEM_SHARED((NACC, H), jnp.float32),
        ],
    )
    def deg_kernel(dst_hbm, ones_hbm, zeros_hbm, out_hbm, dst_iv, ones_v, acc):
        c = lax.axis_index("c")
        s = lax.axis_index("s")
        tile = c * _NS + s
        pltpu.sync_copy(zeros_hbm.at[pl.ds(s * rpt_acc, rpt_acc)],
                        acc.at[pl.ds(s * rpt_acc, rpt_acc)])
        pltpu.sync_copy(ones_hbm, ones_v)
        pltpu.sync_copy(dst_hbm.at[pl.ds(tile * KCHD, KCHD)], dst_iv)
        plsc.subcore_barrier()

        def body(k, carry):
            pltpu.sync_copy(ones_v, acc.at[dst_iv.at[k]], add=True)
            return carry

        lax.fori_loop(0, KCHD, body, 0)
        plsc.subcore_barrier()
        pltpu.sync_copy(acc.at[pl.ds(s * rpt, rpt)],
                        out_hbm.at[c].at[pl.ds(s * rpt, rpt)])

        @pl.when(s == _NS - 1)
        def _():
            pltpu.sync_copy(acc.at[pl.ds(_NS * rpt, tail)],
                            out_hbm.at[c].at[pl.ds(_NS * rpt, tail)])

    return deg_kernel


def _build_sc_spmm(N, NACC, H, KCH):
    rpt, tail = _init_slices(N)
    dums = NACC - N
    IDH = KCH // 4

    @functools.partial(
        pl.kernel,
        out_type=jax.ShapeDtypeStruct((N, H), jnp.float32),
        mesh=_sc_mesh(),
        scratch_types=[
            pltpu.VMEM((IDH, _CH), jnp.int32),
            pltpu.VMEM((IDH, _CH), jnp.int32),
            pltpu.VMEM((2, _CH, H), jnp.float32),
            pltpu.VMEM_SHARED((NACC, H), jnp.float32),
            pltpu.SemaphoreType.DMA((2,)),
        ],
    )
    def spmm_kernel(xw_hbm, zeros_hbm, src_hbm, dst_hbm, out_hbm,
                    src_iv, dst_iv, rows_v, acc, gsem):
        c = lax.axis_index("c")
        s = lax.axis_index("s")

        @pl.when(c == 0)
        def _():
            pltpu.sync_copy(xw_hbm.at[pl.ds(s * rpt, rpt)],
                            acc.at[pl.ds(s * rpt, rpt)])

            @pl.when(s == _NS - 1)
            def _():
                pltpu.sync_copy(xw_hbm.at[pl.ds(_NS * rpt, tail)],
                                acc.at[pl.ds(_NS * rpt, tail)])

            @pl.when(s == 0)
            def _():
                pltpu.sync_copy(zeros_hbm.at[pl.ds(0, dums)],
                                acc.at[pl.ds(N, dums)])

            plsc.subcore_barrier()

            for hh in range(4):
                pltpu.sync_copy(
                    src_hbm.at[pl.ds(s * KCH + hh * IDH, IDH)], src_iv)
                pltpu.sync_copy(
                    dst_hbm.at[pl.ds(s * KCH + hh * IDH, IDH)], dst_iv)
                pltpu.async_copy(xw_hbm.at[src_iv.at[0]], rows_v.at[0],
                                 gsem.at[0])

                def body(k, carry):
                    b = lax.rem(k, 2)
                    pltpu.make_async_copy(xw_hbm.at[src_iv.at[k]],
                                          rows_v.at[b], gsem.at[b]).wait()

                    @pl.when(k + 1 < IDH)
                    def _():
                        pltpu.async_copy(xw_hbm.at[src_iv.at[k + 1]],
                                         rows_v.at[1 - b], gsem.at[1 - b])

                    pltpu.sync_copy(rows_v.at[b], acc.at[dst_iv.at[k]],
                                    add=True)
                    return carry

                lax.fori_loop(0, IDH, body, 0)

            plsc.subcore_barrier()
            pltpu.sync_copy(acc.at[pl.ds(s * rpt, rpt)],
                            out_hbm.at[pl.ds(s * rpt, rpt)])

            @pl.when(s == _NS - 1)
            def _():
                pltpu.sync_copy(acc.at[pl.ds(_NS * rpt, tail)],
                                out_hbm.at[pl.ds(_NS * rpt, tail)])

    return spmm_kernel


def _bn_relu(h, g, b):
    mu = jnp.mean(h, axis=0, keepdims=True)
    d = h - mu
    var = jnp.mean(d * d, axis=0, keepdims=True)
    return jnp.maximum(d * lax.rsqrt(var + 1e-5) * g + b, 0.0)


def _tc_pre_body(x, w_in, b_in, g_in, beta_in, dinv, w0, h_o, xw_o):
    h = jnp.dot(x[...], w_in[...], preferred_element_type=jnp.float32) + b_in[...]
    h = _bn_relu(h, g_in[...], beta_in[...])
    h_o[...] = h
    xw_o[...] = dinv[...] * jnp.dot(h, w0[...], preferred_element_type=jnp.float32)


def _tc_mid_body(sp, dinv, cb, g, b, hres, wn, h_o, xw_o):
    t = dinv[...] * sp[...] + cb[...]
    h = _bn_relu(t, g[...], b[...]) + hres[...]
    h_o[...] = h
    xw_o[...] = dinv[...] * jnp.dot(h, wn[...], preferred_element_type=jnp.float32)


def _tc_post_body(sp, dinv, cb, g, b, hres, wh, bh, out_o):
    t = dinv[...] * sp[...] + cb[...]
    h = _bn_relu(t, g[...], b[...]) + hres[...]
    heads = jnp.dot(h, wh[...], preferred_element_type=jnp.float32) + bh[...]
    clipped = jnp.clip(heads, -10.0, 10.0)
    col = lax.broadcasted_iota(jnp.int32, heads.shape, 1)
    out_o[...] = jnp.where(col == 1, clipped, heads)


def kernel(x, edge_index, W_in, b_in, g_in, beta_in, conv_W, conv_b,
           bn_g, bn_b, W_mean, b_mean, W_lv, b_lv):
    N, _ = x.shape
    H = W_in.shape[1]
    L = conv_W.shape[0]
    E = edge_index.shape[1]
    TPW = _NC * _NS
    KCHT = -(-(-(-E // _CH) // _NS) // 32) * 32
    EP = KCHT * _NS * _CH
    NACC = ((N + 16 * 8) // (16 * 8)) * (16 * 8)

    f32 = jnp.float32
    src = edge_index[0].astype(jnp.int32)
    dst = edge_index[1].astype(jnp.int32)
    pad = EP - E
    src2 = jnp.concatenate(
        [src, jnp.zeros((pad,), jnp.int32)]).reshape(_NS * KCHT, _CH)
    dst2 = jnp.concatenate(
        [dst, jnp.full((pad,), N, jnp.int32)]).reshape(_NS * KCHT, _CH)

    zeros_h = jnp.zeros((NACC, H), f32)
    ones_h = jnp.zeros((_CH, H), f32).at[:, 0].set(1.0)

    degp = _build_sc_deg(N, NACC, H, _NS * KCHT // TPW)(dst2, ones_h, zeros_h)
    deg = degp[0, :, 0] + degp[1, :, 0] + 1.0
    dinv = lax.rsqrt(deg).reshape(N, 1)

    sds = jax.ShapeDtypeStruct
    b2 = lambda v: v.reshape(1, -1)

    h, xw = pl.pallas_call(
        _tc_pre_body, out_shape=(sds((N, H), f32), sds((N, H), f32)))(
            x, W_in, b2(b_in), b2(g_in), b2(beta_in), dinv, conv_W[0])

    spmm = _build_sc_spmm(N, NACC, H, KCHT)
    heads = None
    for i in range(L):
        sp = spmm(xw, zeros_h, src2, dst2)
        if i + 1 < L:
            h, xw = pl.pallas_call(
                _tc_mid_body, out_shape=(sds((N, H), f32), sds((N, H), f32)))(
                    sp, dinv, b2(conv_b[i]), b2(bn_g[i]), b2(bn_b[i]), h,
                    conv_W[i + 1])
        else:
            wh = jnp.concatenate([W_mean, W_lv], axis=1)
            bh = jnp.concatenate([b_mean, b_lv]).reshape(1, 2)
            heads = pl.pallas_call(
                _tc_post_body, out_shape=sds((N, 2), f32))(
                    sp, dinv, b2(conv_b[i]), b2(bn_g[i]), b2(bn_b[i]), h, wh, bh)
    return heads[:, 0], heads[:, 1]


# speedup vs baseline: 1.0445x; 1.0445x over previous
import functools

import jax
import jax.numpy as jnp
from jax import lax
from jax.experimental import pallas as pl
from jax.experimental.pallas import tpu as pltpu
from jax.experimental.pallas import tpu_sc as plsc

_NC = 2
_NS = 16
_CH = 128


def _sc_mesh():
    return plsc.VectorSubcoreMesh(core_axis_name="c", subcore_axis_name="s")


def _init_slices(N):
    rpt = (N // _NS) & ~7
    tail = N - _NS * rpt
    return rpt, tail


def _build_sc_deg(N, NACC, H, KCHD):
    rpt_acc = NACC // _NS
    rpt, tail = _init_slices(N)

    @functools.partial(
        pl.kernel,
        out_type=jax.ShapeDtypeStruct((_NC, N, H), jnp.float32),
        mesh=_sc_mesh(),
        scratch_types=[
            pltpu.VMEM((KCHD, _CH), jnp.int32),
            pltpu.VMEM((_CH, H), jnp.float32),
            pltpu.VMEM_SHARED((NACC, H), jnp.float32),
        ],
    )
    def deg_kernel(dst_hbm, ones_hbm, zeros_hbm, out_hbm, dst_iv, ones_v, acc):
        c = lax.axis_index("c")
        s = lax.axis_index("s")
        tile = c * _NS + s
        pltpu.sync_copy(zeros_hbm.at[pl.ds(s * rpt_acc, rpt_acc)],
                        acc.at[pl.ds(s * rpt_acc, rpt_acc)])
        pltpu.sync_copy(ones_hbm, ones_v)
        pltpu.sync_copy(dst_hbm.at[pl.ds(tile * KCHD, KCHD)], dst_iv)
        plsc.subcore_barrier()

        def body(k, carry):
            pltpu.sync_copy(ones_v, acc.at[dst_iv.at[k]], add=True)
            return carry

        lax.fori_loop(0, KCHD, body, 0)
        plsc.subcore_barrier()
        pltpu.sync_copy(acc.at[pl.ds(s * rpt, rpt)],
                        out_hbm.at[c].at[pl.ds(s * rpt, rpt)])

        @pl.when(s == _NS - 1)
        def _():
            pltpu.sync_copy(acc.at[pl.ds(_NS * rpt, tail)],
                            out_hbm.at[c].at[pl.ds(_NS * rpt, tail)])

    return deg_kernel


def _build_sc_spmm(N, NACC, H, KCH):
    rpt, tail = _init_slices(N)
    rpt1 = NACC // _NS
    dums = NACC - N

    @functools.partial(
        pl.kernel,
        out_type=jax.ShapeDtypeStruct((_NC, N, H), jnp.float32),
        mesh=_sc_mesh(),
        scratch_types=[
            pltpu.VMEM((KCH, _CH), jnp.int32),
            pltpu.VMEM((KCH, _CH), jnp.int32),
            pltpu.VMEM((_CH, H), jnp.float32),
            pltpu.VMEM_SHARED((NACC, H), jnp.float32),
            pltpu.SemaphoreType.DMA,
        ],
    )
    def spmm_kernel(xw_hbm, zeros_hbm, src_hbm, dst_hbm, out_hbm,
                    src_iv, dst_iv, rows_v, acc, gsem):
        c = lax.axis_index("c")
        s = lax.axis_index("s")
        tile = c * _NS + s

        @pl.when(c == 0)
        def _():
            pltpu.sync_copy(xw_hbm.at[pl.ds(s * rpt, rpt)],
                            acc.at[pl.ds(s * rpt, rpt)])

            @pl.when(s == _NS - 1)
            def _():
                pltpu.sync_copy(xw_hbm.at[pl.ds(_NS * rpt, tail)],
                                acc.at[pl.ds(_NS * rpt, tail)])

            @pl.when(s == 0)
            def _():
                pltpu.sync_copy(zeros_hbm.at[pl.ds(0, dums)],
                                acc.at[pl.ds(N, dums)])

        @pl.when(c == 1)
        def _():
            pltpu.sync_copy(zeros_hbm.at[pl.ds(s * rpt1, rpt1)],
                            acc.at[pl.ds(s * rpt1, rpt1)])

        pltpu.sync_copy(src_hbm.at[pl.ds(tile * KCH, KCH)], src_iv)
        pltpu.sync_copy(dst_hbm.at[pl.ds(tile * KCH, KCH)], dst_iv)
        plsc.subcore_barrier()

        def body(k, carry):
            pltpu.async_copy(xw_hbm.at[src_iv.at[k]], rows_v, gsem).wait()
            pltpu.sync_copy(rows_v, acc.at[dst_iv.at[k]], add=True)
            return carry

        lax.fori_loop(0, KCH, body, 0)
        plsc.subcore_barrier()
        pltpu.sync_copy(acc.at[pl.ds(s * rpt, rpt)],
                        out_hbm.at[c].at[pl.ds(s * rpt, rpt)])

        @pl.when(s == _NS - 1)
        def _():
            pltpu.sync_copy(acc.at[pl.ds(_NS * rpt, tail)],
                            out_hbm.at[c].at[pl.ds(_NS * rpt, tail)])

    return spmm_kernel


def _bn_relu(h, g, b):
    mu = jnp.mean(h, axis=0, keepdims=True)
    d = h - mu
    var = jnp.mean(d * d, axis=0, keepdims=True)
    return jnp.maximum(d * lax.rsqrt(var + 1e-5) * g + b, 0.0)


def _tc_pre_body(x, w_in, b_in, g_in, beta_in, dinv, w0, h_o, xw_o):
    h = jnp.dot(x[...], w_in[...], preferred_element_type=jnp.float32) + b_in[...]
    h = _bn_relu(h, g_in[...], beta_in[...])
    h_o[...] = h
    xw_o[...] = dinv[...] * jnp.dot(h, w0[...], preferred_element_type=jnp.float32)


def _tc_mid_body(sp, dinv, cb, g, b, hres, wn, h_o, xw_o):
    t = dinv[...] * (sp[0] + sp[1]) + cb[...]
    h = _bn_relu(t, g[...], b[...]) + hres[...]
    h_o[...] = h
    xw_o[...] = dinv[...] * jnp.dot(h, wn[...], preferred_element_type=jnp.float32)


def _tc_post_body(sp, dinv, cb, g, b, hres, wh, bh, out_o):
    t = dinv[...] * (sp[0] + sp[1]) + cb[...]
    h = _bn_relu(t, g[...], b[...]) + hres[...]
    heads = jnp.dot(h, wh[...], preferred_element_type=jnp.float32) + bh[...]
    clipped = jnp.clip(heads, -10.0, 10.0)
    col = lax.broadcasted_iota(jnp.int32, heads.shape, 1)
    out_o[...] = jnp.where(col == 1, clipped, heads)


def kernel(x, edge_index, W_in, b_in, g_in, beta_in, conv_W, conv_b,
           bn_g, bn_b, W_mean, b_mean, W_lv, b_lv):
    N, _ = x.shape
    H = W_in.shape[1]
    L = conv_W.shape[0]
    E = edge_index.shape[1]
    TPW = _NC * _NS
    KCHT = -(-(-(-E // _CH) // _NS) // 32) * 32
    EP = KCHT * _NS * _CH
    NACC = ((N + 16 * 8) // (16 * 8)) * (16 * 8)

    f32 = jnp.float32
    src = edge_index[0].astype(jnp.int32)
    dst = edge_index[1].astype(jnp.int32)
    pad = EP - E
    src2 = jnp.concatenate(
        [src, jnp.zeros((pad,), jnp.int32)]).reshape(_NS * KCHT, _CH)
    dst2 = jnp.concatenate(
        [dst, jnp.full((pad,), N, jnp.int32)]).reshape(_NS * KCHT, _CH)

    zeros_h = jnp.zeros((NACC, H), f32)
    ones_h = jnp.zeros((_CH, H), f32).at[:, 0].set(1.0)

    degp = _build_sc_deg(N, NACC, H, _NS * KCHT // TPW)(dst2, ones_h, zeros_h)
    deg = degp[0, :, 0] + degp[1, :, 0] + 1.0
    dinv = lax.rsqrt(deg).reshape(N, 1)

    sds = jax.ShapeDtypeStruct
    b2 = lambda v: v.reshape(1, -1)

    h, xw = pl.pallas_call(
        _tc_pre_body, out_shape=(sds((N, H), f32), sds((N, H), f32)))(
            x, W_in, b2(b_in), b2(g_in), b2(beta_in), dinv, conv_W[0])

    spmm = _build_sc_spmm(N, NACC, H, KCHT // 2)
    heads = None
    for i in range(L):
        sp = spmm(xw, zeros_h, src2, dst2)
        if i + 1 < L:
            h, xw = pl.pallas_call(
                _tc_mid_body, out_shape=(sds((N, H), f32), sds((N, H), f32)))(
                    sp, dinv, b2(conv_b[i]), b2(bn_g[i]), b2(bn_b[i]), h,
                    conv_W[i + 1])
        else:
            wh = jnp.concatenate([W_mean, W_lv], axis=1)
            bh = jnp.concatenate([b_mean, b_lv]).reshape(1, 2)
            heads = pl.pallas_call(
                _tc_post_body, out_shape=sds((N, 2), f32))(
                    sp, dinv, b2(conv_b[i]), b2(bn_g[i]), b2(bn_b[i]), h, wh, bh)
    return heads[:, 0], heads[:, 1]


# speedup vs baseline: 1.3517x; 1.2940x over previous
import functools

import jax
import jax.numpy as jnp
from jax import lax
from jax.experimental import pallas as pl
from jax.experimental.pallas import tpu as pltpu
from jax.experimental.pallas import tpu_sc as plsc

_NC = 2
_NS = 16
_CH = 128


def _sc_mesh():
    return plsc.VectorSubcoreMesh(core_axis_name="c", subcore_axis_name="s")


def _init_slices(N):
    rpt = (N // _NS) & ~7
    tail = N - _NS * rpt
    return rpt, tail


def _build_sc_deg(N, NACC, H, KCHD):
    rpt_acc = NACC // _NS
    rpt, tail = _init_slices(N)

    @functools.partial(
        pl.kernel,
        out_type=jax.ShapeDtypeStruct((_NC, N, H), jnp.float32),
        mesh=_sc_mesh(),
        scratch_types=[
            pltpu.VMEM((KCHD, _CH), jnp.int32),
            pltpu.VMEM((_CH, H), jnp.float32),
            pltpu.VMEM_SHARED((NACC, H), jnp.float32),
        ],
    )
    def deg_kernel(dst_hbm, ones_hbm, zeros_hbm, out_hbm, dst_iv, ones_v, acc):
        c = lax.axis_index("c")
        s = lax.axis_index("s")
        tile = c * _NS + s
        pltpu.sync_copy(zeros_hbm.at[pl.ds(s * rpt_acc, rpt_acc)],
                        acc.at[pl.ds(s * rpt_acc, rpt_acc)])
        pltpu.sync_copy(ones_hbm, ones_v)
        pltpu.sync_copy(dst_hbm.at[tile], dst_iv)
        plsc.subcore_barrier()

        def body(k, carry):
            pltpu.sync_copy(ones_v, acc.at[dst_iv.at[k]], add=True)
            return carry

        lax.fori_loop(0, KCHD, body, 0)
        plsc.subcore_barrier()
        pltpu.sync_copy(acc.at[pl.ds(s * rpt, rpt)],
                        out_hbm.at[c].at[pl.ds(s * rpt, rpt)])

        @pl.when(s == _NS - 1)
        def _():
            pltpu.sync_copy(acc.at[pl.ds(_NS * rpt, tail)],
                            out_hbm.at[c].at[pl.ds(_NS * rpt, tail)])

    return deg_kernel


def _build_sc_spmm(N, NACC, H, KCH):
    rpt, tail = _init_slices(N)
    rpt1 = NACC // _NS
    dums = NACC - N

    @functools.partial(
        pl.kernel,
        out_type=jax.ShapeDtypeStruct((_NC, N, H), jnp.float32),
        mesh=_sc_mesh(),
        scratch_types=[
            pltpu.VMEM((KCH, _CH), jnp.int32),
            pltpu.VMEM((KCH, _CH), jnp.int32),
            pltpu.VMEM((_CH, H), jnp.float32),
            pltpu.VMEM_SHARED((NACC, H), jnp.float32),
            pltpu.SemaphoreType.DMA,
        ],
    )
    def spmm_kernel(xw_hbm, zeros_hbm, src_hbm, dst_hbm, out_hbm,
                    src_iv, dst_iv, rows_v, acc, gsem):
        c = lax.axis_index("c")
        s = lax.axis_index("s")
        tile = c * _NS + s

        @pl.when(c == 0)
        def _():
            pltpu.sync_copy(xw_hbm.at[pl.ds(s * rpt, rpt)],
                            acc.at[pl.ds(s * rpt, rpt)])

            @pl.when(s == _NS - 1)
            def _():
                pltpu.sync_copy(xw_hbm.at[pl.ds(_NS * rpt, tail)],
                                acc.at[pl.ds(_NS * rpt, tail)])

            @pl.when(s == 0)
            def _():
                pltpu.sync_copy(zeros_hbm.at[pl.ds(0, dums)],
                                acc.at[pl.ds(N, dums)])

        @pl.when(c == 1)
        def _():
            pltpu.sync_copy(zeros_hbm.at[pl.ds(s * rpt1, rpt1)],
                            acc.at[pl.ds(s * rpt1, rpt1)])

        pltpu.sync_copy(src_hbm.at[tile], src_iv)
        pltpu.sync_copy(dst_hbm.at[tile], dst_iv)
        plsc.subcore_barrier()

        def body(k, carry):
            pltpu.async_copy(xw_hbm.at[src_iv.at[k]], rows_v, gsem).wait()
            pltpu.sync_copy(rows_v, acc.at[dst_iv.at[k]], add=True)
            return carry

        lax.fori_loop(0, KCH, body, 0)
        plsc.subcore_barrier()
        pltpu.sync_copy(acc.at[pl.ds(s * rpt, rpt)],
                        out_hbm.at[c].at[pl.ds(s * rpt, rpt)])

        @pl.when(s == _NS - 1)
        def _():
            pltpu.sync_copy(acc.at[pl.ds(_NS * rpt, tail)],
                            out_hbm.at[c].at[pl.ds(_NS * rpt, tail)])

    return spmm_kernel


def _bn_relu(h, g, b):
    mu = jnp.mean(h, axis=0, keepdims=True)
    d = h - mu
    var = jnp.mean(d * d, axis=0, keepdims=True)
    return jnp.maximum(d * lax.rsqrt(var + 1e-5) * g + b, 0.0)


def _tc_pre_body(x, w_in, b_in, g_in, beta_in, dinv, w0, h_o, xw_o):
    h = jnp.dot(x[...], w_in[...], preferred_element_type=jnp.float32) + b_in[...]
    h = _bn_relu(h, g_in[...], beta_in[...])
    h_o[...] = h
    xw_o[...] = dinv[...] * jnp.dot(h, w0[...], preferred_element_type=jnp.float32)


def _tc_mid_body(sp, dinv, cb, g, b, hres, wn, h_o, xw_o):
    t = dinv[...] * (sp[0] + sp[1]) + cb[...]
    h = _bn_relu(t, g[...], b[...]) + hres[...]
    h_o[...] = h
    xw_o[...] = dinv[...] * jnp.dot(h, wn[...], preferred_element_type=jnp.float32)


def _tc_post_body(sp, dinv, cb, g, b, hres, wh, bh, out_o):
    t = dinv[...] * (sp[0] + sp[1]) + cb[...]
    h = _bn_relu(t, g[...], b[...]) + hres[...]
    heads = jnp.dot(h, wh[...], preferred_element_type=jnp.float32) + bh[...]
    clipped = jnp.clip(heads, -10.0, 10.0)
    col = lax.broadcasted_iota(jnp.int32, heads.shape, 1)
    out_o[...] = jnp.where(col == 1, clipped, heads)


def kernel(x, edge_index, W_in, b_in, g_in, beta_in, conv_W, conv_b,
           bn_g, bn_b, W_mean, b_mean, W_lv, b_lv):
    N, _ = x.shape
    H = W_in.shape[1]
    L = conv_W.shape[0]
    E = edge_index.shape[1]
    TPW = _NC * _NS
    grp = TPW * _CH
    EP = ((E + grp - 1) // grp) * grp
    KCH = EP // grp
    NACC = ((N + 16 * 8) // (16 * 8)) * (16 * 8)

    f32 = jnp.float32
    src = edge_index[0].astype(jnp.int32)
    dst = edge_index[1].astype(jnp.int32)
    pad = EP - E
    src3 = jnp.concatenate(
        [src, jnp.zeros((pad,), jnp.int32)]).reshape(TPW, KCH, _CH)
    dst3 = jnp.concatenate(
        [dst, jnp.full((pad,), N, jnp.int32)]).reshape(TPW, KCH, _CH)

    zeros_h = jnp.zeros((NACC, H), f32)
    ones_h = jnp.zeros((_CH, H), f32).at[:, 0].set(1.0)

    degp = _build_sc_deg(N, NACC, H, KCH)(dst3, ones_h, zeros_h)
    deg = degp[0, :, 0] + degp[1, :, 0] + 1.0
    dinv = lax.rsqrt(deg).reshape(N, 1)

    sds = jax.ShapeDtypeStruct
    b2 = lambda v: v.reshape(1, -1)

    h, xw = pl.pallas_call(
        _tc_pre_body, out_shape=(sds((N, H), f32), sds((N, H), f32)))(
            x, W_in, b2(b_in), b2(g_in), b2(beta_in), dinv, conv_W[0])

    spmm = _build_sc_spmm(N, NACC, H, KCH)
    heads = None
    for i in range(L):
        sp = spmm(xw, zeros_h, src3, dst3)
        if i + 1 < L:
            h, xw = pl.pallas_call(
                _tc_mid_body, out_shape=(sds((N, H), f32), sds((N, H), f32)))(
                    sp, dinv, b2(conv_b[i]), b2(bn_g[i]), b2(bn_b[i]), h,
                    conv_W[i + 1])
        else:
            wh = jnp.concatenate([W_mean, W_lv], axis=1)
            bh = jnp.concatenate([b_mean, b_lv]).reshape(1, 2)
            heads = pl.pallas_call(
                _tc_post_body, out_shape=sds((N, 2), f32))(
                    sp, dinv, b2(conv_b[i]), b2(bn_g[i]), b2(bn_b[i]), h, wh, bh)
    return heads[:, 0], heads[:, 1]
